# baseline (speedup 1.0000x reference)
import jax
import jax.numpy as jnp
from jax import lax
from jax.experimental import pallas as pl
from jax.experimental.pallas import tpu as pltpu
from jax.experimental.pallas import tpu_sc as plsc

HIDDEN = 128
BATCH = 1024
SEQ = 200
SEQ_PAD = 208
SPLIT0 = 128
SPLIT1 = SEQ - SPLIT0
NC = 2
NS = 16
NW = NC * NS
ROWS_PER_W = BATCH // NW
NVH = HIDDEN // 16
NGRP = SEQ_PAD // 16
NSLOT = 3
EPS = 1e-6


def _rsqrt_newton(x):
    xi = lax.bitcast_convert_type(x, jnp.int32)
    yi = jnp.int32(0x5F3759DF) - lax.shift_right_logical(xi, 1)
    y = lax.bitcast_convert_type(yi, jnp.float32)
    xh = x * 0.5
    for _ in range(2):
        y = y * (1.5 - xh * y * y)
    return y


def _body(ids_hbm, tt_hbm, pid_hbm, word_hbm, pos_hbm, typ_hbm, gam_hbm,
          bet_hbm, out_hbm, idx_v, typ_v, pid_v, pos_buf, typ_rows, gam_v,
          bet_v, word_rows, gsem0, gsem1, gsem2, osem0, osem1, osem2, psem):
    c = lax.axis_index("c")
    s = lax.axis_index("s")
    wid = s * NC + c
    r0 = wid * ROWS_PER_W
    gsems = (gsem0, gsem1, gsem2)
    osems = (osem0, osem1, osem2)

    pltpu.sync_copy(pid_hbm.at[0], pid_v)
    pltpu.sync_copy(typ_hbm, typ_rows)
    pltpu.sync_copy(gam_hbm, gam_v)
    pltpu.sync_copy(bet_hbm, bet_v)
    p0 = pltpu.async_copy(pos_hbm.at[pid_v.at[pl.ds(0, SPLIT0)]],
                          pos_buf.at[pl.ds(0, SPLIT0)], psem)
    p1 = pltpu.async_copy(pos_hbm.at[pid_v.at[pl.ds(SPLIT0, SPLIT1)]],
                          pos_buf.at[pl.ds(SPLIT0, SPLIT1)], psem)
    p0.wait()
    p1.wait()

    def fold_body(t, carry):
        for h in range(NVH):
            sl = pl.ds(h * 16, 16)
            pos_buf[t, sl] = pos_buf[t, sl] + typ_rows[0, sl]
        return carry
    lax.fori_loop(0, SEQ, fold_body, 0, unroll=False)

    def ids_start(slot, r):
        pltpu.async_copy(ids_hbm.at[r], idx_v.at[slot, pl.ds(0, SEQ)], psem)
        pltpu.async_copy(tt_hbm.at[r], typ_v.at[slot, pl.ds(0, SEQ)], psem)

    def ids_wait(slot):
        pltpu.make_async_copy(ids_hbm.at[0], idx_v.at[slot, pl.ds(0, SEQ)],
                              psem).wait()
        pltpu.make_async_copy(tt_hbm.at[0], typ_v.at[slot, pl.ds(0, SEQ)],
                              psem).wait()

    def stage(slot, r):
        del r
        for off, ln in ((0, 24), (24, 24), (48, 24), (72, 24), (96, 24),
                        (120, 24), (144, 24), (168, 32)):
            pltpu.async_copy(word_hbm.at[idx_v.at[slot, pl.ds(off, ln)]],
                             word_rows.at[slot, pl.ds(off, ln)], gsems[slot])

    def wait_gather(slot):
        pltpu.make_async_copy(word_hbm.at[pl.ds(0, SEQ)],
                              word_rows.at[slot, pl.ds(0, SEQ)],
                              gsems[slot]).wait()

    def start_out(slot, r):
        pltpu.async_copy(word_rows.at[slot, pl.ds(0, SEQ)], out_hbm.at[r],
                         osems[slot])

    def wait_out(slot):
        pltpu.make_async_copy(word_rows.at[slot, pl.ds(0, SEQ)],
                              out_hbm.at[0], osems[slot]).wait()

    def compute(slot):
        def one_token(t, tfs, dv, gv, bv):
            tf = jnp.full((16,), tfs, jnp.float32)
            ws = []
            for h in range(NVH):
                sl = pl.ds(h * 16, 16)
                w = word_rows[slot, t, sl] + pos_buf[t, sl] + tf * dv[h]
                ws.append(w)
            ssum = ((ws[0] + ws[1]) + (ws[2] + ws[3])) + \
                   ((ws[4] + ws[5]) + (ws[6] + ws[7]))
            qsum = ((ws[0] * ws[0] + ws[1] * ws[1]) +
                    (ws[2] * ws[2] + ws[3] * ws[3])) + \
                   ((ws[4] * ws[4] + ws[5] * ws[5]) +
                    (ws[6] * ws[6] + ws[7] * ws[7]))
            stot = jnp.sum(ssum)
            qtot = jnp.sum(qsum)
            mean = stot * (1.0 / HIDDEN)
            var = qtot * (1.0 / HIDDEN) - mean * mean
            varv = jnp.full((16,), var + EPS, jnp.float32)
            rstd = _rsqrt_newton(varv)
            meanv = jnp.full((16,), mean, jnp.float32)
            for h in range(NVH):
                sl = pl.ds(h * 16, 16)
                word_rows[slot, t, sl] = \
                    ((ws[h] - meanv) * rstd) * gv[h] + bv[h]

        dvecs = tuple(typ_rows[1, pl.ds(h * 16, 16)] -
                      typ_rows[0, pl.ds(h * 16, 16)] for h in range(NVH))
        gvecs = tuple(gam_v[pl.ds(h * 16, 16)] for h in range(NVH))
        bvecs = tuple(bet_v[pl.ds(h * 16, 16)] for h in range(NVH))

        def grp_body(g, carry2):
            dv, gv, bv = carry2
            base = g * 16
            tvf = typ_v[slot, pl.ds(base, 16)].astype(jnp.float32)
            for j in range(16):
                one_token(base + j, tvf[j], dv, gv, bv)
            return (dv, gv, bv)

        lax.fori_loop(0, NGRP, grp_body, (dvecs, gvecs, bvecs),
                      unroll=False)

    ids_start(0, r0)
    ids_wait(0)
    ids_start(1, r0 + 1)
    stage(0, r0)
    ids_wait(1)
    stage(1, r0 + 1)

    def triple_body(p, carry):
        x = r0 + 3 * p
        for k in range(3):
            nslot = (k + 2) % 3
            ids_start(nslot, x + k + 2)
            wait_gather(k)
            compute(k)
            start_out(k, x + k)

            if k == 0:
                @pl.when(p > 0)
                def _():
                    wait_out(nslot)
            else:
                wait_out(nslot)
            ids_wait(nslot)
            stage(nslot, x + k + 2)
        return carry

    lax.fori_loop(0, ROWS_PER_W // 3, triple_body, 0, unroll=False)
    wait_gather(0)
    compute(0)
    start_out(0, r0 + ROWS_PER_W - 2)
    wait_gather(1)
    compute(1)
    start_out(1, r0 + ROWS_PER_W - 1)
    wait_out(2)
    wait_out(0)
    wait_out(1)


@jax.jit
def _bert_embed(ids, tts, pid, word, pos, typ, gam, bet):
    mesh = plsc.VectorSubcoreMesh(core_axis_name="c", subcore_axis_name="s")
    fn = pl.kernel(
        _body,
        out_type=jax.ShapeDtypeStruct((BATCH, SEQ, HIDDEN), jnp.float32),
        mesh=mesh,
        compiler_params=pltpu.CompilerParams(needs_layout_passes=False,
                                             use_tc_tiling_on_sc=False),
        scratch_types=[
            pltpu.VMEM((NSLOT, SEQ_PAD), jnp.int32),
            pltpu.VMEM((NSLOT, SEQ_PAD), jnp.int32),
            pltpu.VMEM((SEQ,), jnp.int32),
            pltpu.VMEM((SEQ, HIDDEN), jnp.float32),
            pltpu.VMEM((2, HIDDEN), jnp.float32),
            pltpu.VMEM((HIDDEN,), jnp.float32),
            pltpu.VMEM((HIDDEN,), jnp.float32),
            pltpu.VMEM((NSLOT, SEQ_PAD, HIDDEN), jnp.float32),
            pltpu.SemaphoreType.DMA,
            pltpu.SemaphoreType.DMA,
            pltpu.SemaphoreType.DMA,
            pltpu.SemaphoreType.DMA,
            pltpu.SemaphoreType.DMA,
            pltpu.SemaphoreType.DMA,
            pltpu.SemaphoreType.DMA,
        ],
    )
    return fn(ids, tts, pid, word, pos, typ, gam, bet)


def kernel(input_ids, token_type_ids, position_ids, attention_mask,
           word_embeddings, position_embeddings, token_type_embeddings,
           gamma, beta):
    del attention_mask
    ids = input_ids.astype(jnp.int32)
    tts = token_type_ids.astype(jnp.int32)
    pid = jnp.atleast_2d(position_ids.astype(jnp.int32))
    return _bert_embed(ids, tts, pid, word_embeddings, position_embeddings,
                       token_type_embeddings, gamma, beta)


# speedup vs baseline: 1.0142x; 1.0142x over previous
import jax
import jax.numpy as jnp
from jax import lax
from jax.experimental import pallas as pl
from jax.experimental.pallas import tpu as pltpu
from jax.experimental.pallas import tpu_sc as plsc

HIDDEN = 128
BATCH = 1024
SEQ = 200
SEQ_PAD = 208
SPLIT0 = 128
SPLIT1 = SEQ - SPLIT0
NC = 2
NS = 16
NW = NC * NS
ROWS_PER_W = BATCH // NW
NVH = HIDDEN // 16
NGRP = SEQ_PAD // 16
NSLOT = 3
EPS = 1e-6


def _rsqrt_newton(x):
    xi = lax.bitcast_convert_type(x, jnp.int32)
    yi = jnp.int32(0x5F3759DF) - lax.shift_right_logical(xi, 1)
    y = lax.bitcast_convert_type(yi, jnp.float32)
    xh = x * 0.5
    for _ in range(2):
        y = y * (1.5 - xh * y * y)
    return y


def _body(ids_hbm, tt_hbm, pid_hbm, word_hbm, pos_hbm, typ_hbm, gam_hbm,
          bet_hbm, out_hbm, idx_v, typ_v, pid_v, pos_buf, typ_rows, gam_v,
          bet_v, word_rows, gsem0, gsem1, gsem2, osem0, osem1, osem2, psem):
    c = lax.axis_index("c")
    s = lax.axis_index("s")
    wid = s * NC + c
    r0 = wid * ROWS_PER_W
    gsems = (gsem0, gsem1, gsem2)
    osems = (osem0, osem1, osem2)

    def ids_start(slot, r):
        pltpu.async_copy(ids_hbm.at[r], idx_v.at[slot, pl.ds(0, SEQ)], psem)
        pltpu.async_copy(tt_hbm.at[r], typ_v.at[slot, pl.ds(0, SEQ)], psem)

    def ids_wait(slot):
        pltpu.make_async_copy(ids_hbm.at[0], idx_v.at[slot, pl.ds(0, SEQ)],
                              psem).wait()
        pltpu.make_async_copy(tt_hbm.at[0], typ_v.at[slot, pl.ds(0, SEQ)],
                              psem).wait()

    def stage(slot, r):
        del r
        for off, ln in ((0, 56), (56, 56), (112, 56), (168, 32)):
            pltpu.async_copy(word_hbm.at[idx_v.at[slot, pl.ds(off, ln)]],
                             word_rows.at[slot, pl.ds(off, ln)], gsems[slot])

    def wait_gather(slot):
        pltpu.make_async_copy(word_hbm.at[pl.ds(0, SEQ)],
                              word_rows.at[slot, pl.ds(0, SEQ)],
                              gsems[slot]).wait()

    def start_out(slot, r):
        pltpu.async_copy(word_rows.at[slot, pl.ds(0, SEQ)], out_hbm.at[r],
                         osems[slot])

    def wait_out(slot):
        pltpu.make_async_copy(word_rows.at[slot, pl.ds(0, SEQ)],
                              out_hbm.at[0], osems[slot]).wait()

    def compute(slot):
        def one_token(t, tfs, dv, gv, bv):
            tf = jnp.full((16,), tfs, jnp.float32)
            ws = []
            for h in range(NVH):
                sl = pl.ds(h * 16, 16)
                w = word_rows[slot, t, sl] + pos_buf[t, sl] + tf * dv[h]
                ws.append(w)
            ssum = ((ws[0] + ws[1]) + (ws[2] + ws[3])) + \
                   ((ws[4] + ws[5]) + (ws[6] + ws[7]))
            qsum = ((ws[0] * ws[0] + ws[1] * ws[1]) +
                    (ws[2] * ws[2] + ws[3] * ws[3])) + \
                   ((ws[4] * ws[4] + ws[5] * ws[5]) +
                    (ws[6] * ws[6] + ws[7] * ws[7]))
            stot = jnp.sum(ssum)
            qtot = jnp.sum(qsum)
            mean = stot * (1.0 / HIDDEN)
            var = qtot * (1.0 / HIDDEN) - mean * mean
            varv = jnp.full((16,), var + EPS, jnp.float32)
            rstd = _rsqrt_newton(varv)
            meanv = jnp.full((16,), mean, jnp.float32)
            for h in range(NVH):
                sl = pl.ds(h * 16, 16)
                word_rows[slot, t, sl] = \
                    ((ws[h] - meanv) * rstd) * gv[h] + bv[h]

        dvecs = tuple(typ_rows[1, pl.ds(h * 16, 16)] -
                      typ_rows[0, pl.ds(h * 16, 16)] for h in range(NVH))
        gvecs = tuple(gam_v[pl.ds(h * 16, 16)] for h in range(NVH))
        bvecs = tuple(bet_v[pl.ds(h * 16, 16)] for h in range(NVH))

        def grp_body(g, carry2):
            dv, gv, bv = carry2
            base = g * 16
            tvf = typ_v[slot, pl.ds(base, 16)].astype(jnp.float32)
            for j in range(16):
                one_token(base + j, tvf[j], dv, gv, bv)
            return (dv, gv, bv)

        lax.fori_loop(0, NGRP, grp_body, (dvecs, gvecs, bvecs),
                      unroll=False)

    pltpu.sync_copy(pid_hbm.at[0], pid_v)
    p0 = pltpu.async_copy(pos_hbm.at[pid_v.at[pl.ds(0, SPLIT0)]],
                          pos_buf.at[pl.ds(0, SPLIT0)], osem2)
    p1 = pltpu.async_copy(pos_hbm.at[pid_v.at[pl.ds(SPLIT0, SPLIT1)]],
                          pos_buf.at[pl.ds(SPLIT0, SPLIT1)], osem2)
    t0 = pltpu.async_copy(typ_hbm, typ_rows, osem2)
    g0 = pltpu.async_copy(gam_hbm, gam_v, osem2)
    b0 = pltpu.async_copy(bet_hbm, bet_v, osem2)
    ids_start(0, r0)
    ids_wait(0)
    ids_start(1, r0 + 1)
    stage(0, r0)
    ids_wait(1)
    stage(1, r0 + 1)
    p0.wait()
    p1.wait()
    t0.wait()
    g0.wait()
    b0.wait()

    def fold_body(t, carry):
        for h in range(NVH):
            sl = pl.ds(h * 16, 16)
            pos_buf[t, sl] = pos_buf[t, sl] + typ_rows[0, sl]
        return carry
    lax.fori_loop(0, SEQ, fold_body, 0, unroll=False)

    def triple_body(p, carry):
        x = r0 + 3 * p
        for k in range(3):
            nslot = (k + 2) % 3
            ids_start(nslot, x + k + 2)
            wait_gather(k)
            compute(k)
            start_out(k, x + k)

            if k == 0:
                @pl.when(p > 0)
                def _():
                    wait_out(nslot)
            else:
                wait_out(nslot)
            ids_wait(nslot)
            stage(nslot, x + k + 2)
        return carry

    lax.fori_loop(0, ROWS_PER_W // 3, triple_body, 0, unroll=False)
    wait_gather(0)
    compute(0)
    start_out(0, r0 + ROWS_PER_W - 2)
    wait_gather(1)
    compute(1)
    start_out(1, r0 + ROWS_PER_W - 1)
    wait_out(2)
    wait_out(0)
    wait_out(1)


@jax.jit
def _bert_embed(ids, tts, pid, word, pos, typ, gam, bet):
    mesh = plsc.VectorSubcoreMesh(core_axis_name="c", subcore_axis_name="s")
    fn = pl.kernel(
        _body,
        out_type=jax.ShapeDtypeStruct((BATCH, SEQ, HIDDEN), jnp.float32),
        mesh=mesh,
        compiler_params=pltpu.CompilerParams(needs_layout_passes=False,
                                             use_tc_tiling_on_sc=False),
        scratch_types=[
            pltpu.VMEM((NSLOT, SEQ_PAD), jnp.int32),
            pltpu.VMEM((NSLOT, SEQ_PAD), jnp.int32),
            pltpu.VMEM((SEQ,), jnp.int32),
            pltpu.VMEM((SEQ, HIDDEN), jnp.float32),
            pltpu.VMEM((2, HIDDEN), jnp.float32),
            pltpu.VMEM((HIDDEN,), jnp.float32),
            pltpu.VMEM((HIDDEN,), jnp.float32),
            pltpu.VMEM((NSLOT, SEQ_PAD, HIDDEN), jnp.float32),
            pltpu.SemaphoreType.DMA,
            pltpu.SemaphoreType.DMA,
            pltpu.SemaphoreType.DMA,
            pltpu.SemaphoreType.DMA,
            pltpu.SemaphoreType.DMA,
            pltpu.SemaphoreType.DMA,
            pltpu.SemaphoreType.DMA,
        ],
    )
    return fn(ids, tts, pid, word, pos, typ, gam, bet)


def kernel(input_ids, token_type_ids, position_ids, attention_mask,
           word_embeddings, position_embeddings, token_type_embeddings,
           gamma, beta):
    del attention_mask
    ids = input_ids.astype(jnp.int32)
    tts = token_type_ids.astype(jnp.int32)
    pid = jnp.atleast_2d(position_ids.astype(jnp.int32))
    return _bert_embed(ids, tts, pid, word_embeddings, position_embeddings,
                       token_type_embeddings, gamma, beta)
